# baseline (speedup 1.0000x reference)
import functools

import jax
import jax.numpy as jnp
from jax import lax
from jax.experimental import pallas as pl
from jax.experimental.pallas import tpu as pltpu
from jax.experimental.pallas import tpu_sc as plsc

_NC = 2
_NS = 16


@functools.lru_cache(maxsize=None)
def _make_sc_broadcast(B, V):
    half = B // _NC
    R = 512
    rows_fill = R // _NS
    n_out = half // (_NS * R)

    mesh = plsc.VectorSubcoreMesh(core_axis_name="c", subcore_axis_name="s")

    @functools.partial(
        pl.kernel,
        out_type=(
            jax.ShapeDtypeStruct((half, V), jnp.float32),
            jax.ShapeDtypeStruct((half, V), jnp.float32),
        ),
        mesh=mesh,
        scratch_types=[
            pltpu.VMEM_SHARED((R, V), jnp.float32),
            pltpu.SemaphoreType.DMA,
        ],
    )
    def broadcast_kernel(table_hbm, out0_hbm, out1_hbm, shared_buf, sem):
        cid = lax.axis_index("c")
        sid = lax.axis_index("s")
        fills = [
            pltpu.async_copy(table_hbm, shared_buf.at[sid * rows_fill + r], sem)
            for r in range(rows_fill)
        ]
        for cp in fills:
            cp.wait()
        plsc.subcore_barrier()
        base = sid * R
        for out_hbm, c_owner in ((out0_hbm, 0), (out1_hbm, 1)):
            @pl.when(cid == c_owner)
            def _():
                copies = [
                    pltpu.async_copy(
                        shared_buf,
                        out_hbm.at[pl.ds(base + c * _NS * R, R)],
                        sem,
                    )
                    for c in range(n_out)
                ]
                for cp in copies:
                    cp.wait()

    return broadcast_kernel


def kernel(x, table):
    B = x.shape[0]
    V = table.shape[0]
    fn = _make_sc_broadcast(B, V)
    a, b = fn(table.reshape(V))
    return jnp.concatenate([a, b], axis=0)


# speedup vs baseline: 1.1256x; 1.1256x over previous
import functools

import jax
import jax.numpy as jnp
from jax import lax
from jax.experimental import pallas as pl
from jax.experimental.pallas import tpu as pltpu
from jax.experimental.pallas import tpu_sc as plsc

_NC = 2
_NS = 16
_NW = _NC * _NS


@functools.lru_cache(maxsize=None)
def _make_sc_broadcast(B, V):
    R = B // _NW
    rows_fill = R // _NS
    flat = R * V

    mesh = plsc.VectorSubcoreMesh(core_axis_name="c", subcore_axis_name="s")

    @functools.partial(
        pl.kernel,
        out_type=jax.ShapeDtypeStruct((B * V,), jnp.float32),
        mesh=mesh,
        scratch_types=[
            pltpu.VMEM((V,), jnp.float32),
            pltpu.VMEM_SHARED((flat,), jnp.float32),
            pltpu.SemaphoreType.DMA,
        ],
    )
    def broadcast_kernel(table_hbm, out_hbm, tab_v, shared_buf, sem):
        cid = lax.axis_index("c")
        sid = lax.axis_index("s")
        pltpu.sync_copy(table_hbm, tab_v)
        fills = [
            pltpu.async_copy(
                tab_v,
                shared_buf.at[pl.ds((sid * rows_fill + r) * V, V)],
                sem,
            )
            for r in range(rows_fill)
        ]
        for cp in fills:
            cp.wait()
        plsc.subcore_barrier()
        wid = cid * _NS + sid
        pltpu.sync_copy(shared_buf, out_hbm.at[pl.ds(wid * flat, flat)])

    return broadcast_kernel


def kernel(x, table):
    B = x.shape[0]
    V = table.shape[0]
    fn = _make_sc_broadcast(B, V)
    return fn(table.reshape(V)).reshape(B, V)
